# baseline (speedup 1.0000x reference)
import jax
import jax.numpy as jnp
from jax import lax
from jax.experimental import pallas as pl
from jax.experimental.pallas import tpu as pltpu
from jax.experimental.pallas import tpu_sc as plsc

GAMMA = 12.0
N_EDGES_TOTAL = 160000
ED = 128
RD = 16
L = 16
NC = 2
NS = 16
NW = NC * NS
G = 16
NGROUPS = N_EDGES_TOTAL // G
TMAX = (NGROUPS + NW - 1) // NW
SM = 17

_GDN = lax.GatherDimensionNumbers(
    offset_dims=(), collapsed_slice_dims=(0,), start_index_map=(0,))


def _bcast_lane(vec, i):
    idx = jnp.full((L, 1), i, jnp.int32)
    return lax.gather(vec, idx, _GDN, slice_sizes=(1,),
                      mode=lax.GatherScatterMode.PROMISE_IN_BOUNDS)


def _body(node_ref, ei0_ref, ei1_ref, rid_ref, rel_ref, proj_ref, out_ref,
          head0, tail0, proj0, sem0,
          head1, tail1, proj1, sem1,
          hidx_a, tidx_a, ridx_a, rel_tab, score_m, out_v):
    wid = lax.axis_index("s") * NC + lax.axis_index("c")
    ngroups = jnp.where(wid < L, TMAX, TMAX - 1)
    g0 = wid * (TMAX - 1) + jnp.minimum(wid, L)
    e0 = g0 * G
    pe0 = jnp.minimum(e0, N_EDGES_TOTAL - TMAX * G)
    off0 = e0 - pe0
    pltpu.sync_copy(ei0_ref.at[pl.ds(pe0, TMAX * G)], hidx_a)
    pltpu.sync_copy(ei1_ref.at[pl.ds(pe0, TMAX * G)], tidx_a)
    pltpu.sync_copy(rid_ref.at[pl.ds(pe0, TMAX * G)], ridx_a)
    pltpu.sync_copy(rel_ref, rel_tab)
    lanes = lax.iota(jnp.int32, L)

    bufs = ((head0, tail0, proj0, sem0), (head1, tail1, proj1, sem1))

    def issue(t, buf):
        head_v, tail_v, proj_v, sem = buf

        @pl.when(t < ngroups)
        def _():
            hv = hidx_a[pl.ds(off0 + t * G, G)]
            tv = tidx_a[pl.ds(off0 + t * G, G)]
            rv = ridx_a[pl.ds(off0 + t * G, G)]
            pltpu.async_copy(node_ref.at[hv], head_v, sem)
            pltpu.async_copy(node_ref.at[tv], tail_v, sem)
            pltpu.async_copy(proj_ref.at[rv], proj_v, sem)

    def compute(t, buf):
        head_v, tail_v, proj_v, sem = buf

        @pl.when(t < ngroups)
        def _():
            pltpu.make_async_copy(node_ref.at[hidx_a.at[pl.ds(0, G)]],
                                  head_v, sem).wait()
            pltpu.make_async_copy(node_ref.at[hidx_a.at[pl.ds(0, G)]],
                                  tail_v, sem).wait()
            pltpu.make_async_copy(proj_ref.at[ridx_a.at[pl.ds(0, G)]],
                                  proj_v, sem).wait()
            rvec = ridx_a[pl.ds(off0 + t * G, G)]

            @plsc.parallel_loop(0, G, 1, unroll=8)
            def edge(e):
                rb = _bcast_lane(rvec, e)
                accs = [plsc.load_gather(rel_tab, [rb * RD + lanes])]
                accs += [jnp.zeros((L,), jnp.float32) for _ in range(7)]
                for c in range(ED // L):
                    dv = (head_v[e, pl.ds(c * L, L)]
                          - tail_v[e, pl.ds(c * L, L)])
                    for m in range(L // 2):
                        pw = proj_v[e, pl.ds((c * (L // 2) + m) * RD, RD)]
                        pa, pb = plsc.unpack(
                            plsc.bitcast(pw, jnp.bfloat16),
                            format=plsc.PackFormat.INTERLEAVED)
                        k = (2 * m) % 8
                        accs[k] = accs[k] + _bcast_lane(dv, 2 * m) * pa
                        accs[k + 1] = (accs[k + 1]
                                       + _bcast_lane(dv, 2 * m + 1) * pb)
                a0 = (accs[0] + accs[2]) + (accs[4] + accs[6])
                a1 = (accs[1] + accs[3]) + (accs[5] + accs[7])
                plsc.store_scatter(
                    score_m, [jnp.full((L,), e * SM, jnp.int32) + lanes],
                    jnp.full((L,), GAMMA / L, jnp.float32)
                    - jnp.abs(a0 + a1))

            rows = lanes * SM
            sv = plsc.load_gather(score_m, [rows])
            for j in range(1, L):
                sv = sv + plsc.load_gather(
                    score_m, [rows + jnp.full((L,), j, jnp.int32)])
            out_v[...] = sv
            pltpu.sync_copy(out_v, out_ref.at[pl.ds(e0 + t * G, G)])

    issue(0, bufs[0])

    def pair(p, carry):
        t = p * 2
        issue(t + 1, bufs[1])
        compute(t, bufs[0])
        issue(t + 2, bufs[0])
        compute(t + 1, bufs[1])
        return carry

    lax.fori_loop(0, (TMAX + 1) // 2, pair, 0, unroll=False)


@jax.jit
def _sc_call(node_emb, edge_index, rel_id, rel_emb_table, projection_table):
    mesh = plsc.VectorSubcoreMesh(core_axis_name="c", subcore_axis_name="s")
    staging = [
        pltpu.VMEM((G, ED), jnp.float32),
        pltpu.VMEM((G, ED), jnp.float32),
        pltpu.VMEM((G, ED * RD // 2), jnp.int32),
        pltpu.SemaphoreType.DMA,
    ]
    f = pl.kernel(
        _body,
        out_type=jax.ShapeDtypeStruct((N_EDGES_TOTAL,), jnp.float32),
        mesh=mesh,
        scratch_types=staging + staging + [
            pltpu.VMEM((TMAX * G,), jnp.int32),
            pltpu.VMEM((TMAX * G,), jnp.int32),
            pltpu.VMEM((TMAX * G,), jnp.int32),
            pltpu.VMEM((1000 * RD,), jnp.float32),
            pltpu.VMEM((G * SM,), jnp.float32),
            pltpu.VMEM((G,), jnp.float32),
        ],
        compiler_params=pltpu.CompilerParams(needs_layout_passes=False),
    )
    proj_bf = (projection_table.reshape(-1, ED // 2, 2, RD)
               .swapaxes(2, 3).astype(jnp.bfloat16)
               .reshape(-1, ED * RD // 2, 2))
    proj_i32 = lax.bitcast_convert_type(proj_bf, jnp.int32)
    return f(node_emb, edge_index[0], edge_index[1], rel_id,
             rel_emb_table.reshape(-1), proj_i32)


def kernel(node_emb, edge_index, rel_id, rel_emb_table, projection_table):
    return _sc_call(node_emb, edge_index, rel_id, rel_emb_table,
                    projection_table)


# speedup vs baseline: 2.4132x; 2.4132x over previous
import jax
import jax.numpy as jnp
from jax import lax
from jax.experimental import pallas as pl
from jax.experimental.pallas import tpu as pltpu
from jax.experimental.pallas import tpu_sc as plsc

GAMMA = 12.0
N_EDGES_TOTAL = 160000
ED = 128
RD = 16
L = 16
NC = 2
NS = 16
NW = NC * NS
G = 16
NGROUPS = N_EDGES_TOTAL // G
TMAX = (NGROUPS + NW - 1) // NW
SM = 17

_GDN = lax.GatherDimensionNumbers(
    offset_dims=(), collapsed_slice_dims=(0,), start_index_map=(0,))


def _bcast_lane(vec, i):
    idx = jnp.full((L, 1), i, jnp.int32)
    return lax.gather(vec, idx, _GDN, slice_sizes=(1,),
                      mode=lax.GatherScatterMode.PROMISE_IN_BOUNDS)


def _body(node_ref, ei0_ref, ei1_ref, rid_ref, rel_ref, proj_ref, out_ref,
          head0, tail0, proj0, sem0,
          head1, tail1, proj1, sem1,
          hidx_a, tidx_a, ridx_a, rel_tab, score_m, out_v):
    wid = lax.axis_index("s") * NC + lax.axis_index("c")
    ngroups = jnp.where(wid < L, TMAX, TMAX - 1)
    g0 = wid * (TMAX - 1) + jnp.minimum(wid, L)
    e0 = g0 * G
    pe0 = jnp.minimum(e0, N_EDGES_TOTAL - TMAX * G)
    off0 = e0 - pe0
    pltpu.sync_copy(ei0_ref.at[pl.ds(pe0, TMAX * G)], hidx_a)
    pltpu.sync_copy(ei1_ref.at[pl.ds(pe0, TMAX * G)], tidx_a)
    pltpu.sync_copy(rid_ref.at[pl.ds(pe0, TMAX * G)], ridx_a)
    pltpu.sync_copy(rel_ref, rel_tab)
    lanes = lax.iota(jnp.int32, L)

    bufs = ((head0, tail0, proj0, sem0), (head1, tail1, proj1, sem1))

    def issue(t, buf):
        head_v, tail_v, proj_v, sem = buf

        @pl.when(t < ngroups)
        def _():
            hv = hidx_a[pl.ds(off0 + t * G, G)]
            tv = tidx_a[pl.ds(off0 + t * G, G)]
            rv = ridx_a[pl.ds(off0 + t * G, G)]
            pltpu.async_copy(node_ref.at[hv], head_v, sem)
            pltpu.async_copy(node_ref.at[tv], tail_v, sem)
            pltpu.async_copy(proj_ref.at[rv], proj_v, sem)

    def compute(t, buf):
        head_v, tail_v, proj_v, sem = buf

        @pl.when(t < ngroups)
        def _():
            pltpu.make_async_copy(node_ref.at[hidx_a.at[pl.ds(0, G)]],
                                  head_v, sem).wait()
            pltpu.make_async_copy(node_ref.at[hidx_a.at[pl.ds(0, G)]],
                                  tail_v, sem).wait()
            pltpu.make_async_copy(proj_ref.at[ridx_a.at[pl.ds(0, G)]],
                                  proj_v, sem).wait()
            rvec = ridx_a[pl.ds(off0 + t * G, G)]

            @plsc.parallel_loop(0, G, 1, unroll=4)
            def edge(e):
                rb = _bcast_lane(rvec, e)
                accs = [plsc.load_gather(rel_tab, [rb * RD + lanes])]
                accs += [jnp.zeros((L,), jnp.float32) for _ in range(7)]
                for c in range(ED // L):
                    dv = (head_v[e, pl.ds(c * L, L)]
                          - tail_v[e, pl.ds(c * L, L)])
                    for m in range(L // 2):
                        pw = proj_v[e, pl.ds((c * (L // 2) + m) * RD, RD)]
                        pa, pb = plsc.unpack(
                            plsc.bitcast(pw, jnp.bfloat16),
                            format=plsc.PackFormat.INTERLEAVED)
                        k = (2 * m) % 8
                        accs[k] = accs[k] + _bcast_lane(dv, 2 * m) * pa
                        accs[k + 1] = (accs[k + 1]
                                       + _bcast_lane(dv, 2 * m + 1) * pb)
                a0 = (accs[0] + accs[2]) + (accs[4] + accs[6])
                a1 = (accs[1] + accs[3]) + (accs[5] + accs[7])
                plsc.store_scatter(
                    score_m, [jnp.full((L,), e * SM, jnp.int32) + lanes],
                    jnp.full((L,), GAMMA / L, jnp.float32)
                    - jnp.abs(a0 + a1))

            rows = lanes * SM
            sv = plsc.load_gather(score_m, [rows])
            for j in range(1, L):
                sv = sv + plsc.load_gather(
                    score_m, [rows + jnp.full((L,), j, jnp.int32)])
            out_v[...] = sv
            pltpu.sync_copy(out_v, out_ref.at[pl.ds(e0 + t * G, G)])

    issue(0, bufs[0])

    def pair(p, carry):
        t = p * 2
        issue(t + 1, bufs[1])
        compute(t, bufs[0])
        issue(t + 2, bufs[0])
        compute(t + 1, bufs[1])
        return carry

    lax.fori_loop(0, (TMAX + 1) // 2, pair, 0, unroll=False)


@jax.jit
def _sc_call(node_emb, edge_index, rel_id, rel_emb_table, projection_table):
    mesh = plsc.VectorSubcoreMesh(core_axis_name="c", subcore_axis_name="s")
    staging = [
        pltpu.VMEM((G, ED), jnp.float32),
        pltpu.VMEM((G, ED), jnp.float32),
        pltpu.VMEM((G, ED * RD // 2), jnp.int32),
        pltpu.SemaphoreType.DMA,
    ]
    f = pl.kernel(
        _body,
        out_type=jax.ShapeDtypeStruct((N_EDGES_TOTAL,), jnp.float32),
        mesh=mesh,
        scratch_types=staging + staging + [
            pltpu.VMEM((TMAX * G,), jnp.int32),
            pltpu.VMEM((TMAX * G,), jnp.int32),
            pltpu.VMEM((TMAX * G,), jnp.int32),
            pltpu.VMEM((1000 * RD,), jnp.float32),
            pltpu.VMEM((G * SM,), jnp.float32),
            pltpu.VMEM((G,), jnp.float32),
        ],
        compiler_params=pltpu.CompilerParams(needs_layout_passes=False),
    )
    proj_bf = (projection_table.reshape(-1, ED // 2, 2, RD)
               .swapaxes(2, 3).astype(jnp.bfloat16)
               .reshape(-1, ED * RD // 2, 2))
    proj_i32 = lax.bitcast_convert_type(proj_bf, jnp.int32)
    return f(node_emb, edge_index[0], edge_index[1], rel_id,
             rel_emb_table.reshape(-1), proj_i32)


def kernel(node_emb, edge_index, rel_id, rel_emb_table, projection_table):
    return _sc_call(node_emb, edge_index, rel_id, rel_emb_table,
                    projection_table)


# speedup vs baseline: 2.4212x; 1.0033x over previous
import jax
import jax.numpy as jnp
from jax import lax
from jax.experimental import pallas as pl
from jax.experimental.pallas import tpu as pltpu
from jax.experimental.pallas import tpu_sc as plsc

GAMMA = 12.0
N_EDGES_TOTAL = 160000
ED = 128
RD = 16
L = 16
NC = 2
NS = 16
NW = NC * NS
G = 16
NGROUPS = N_EDGES_TOTAL // G
TMAX = (NGROUPS + NW - 1) // NW
SM = 17

_GDN = lax.GatherDimensionNumbers(
    offset_dims=(), collapsed_slice_dims=(0,), start_index_map=(0,))


def _bcast_lane(vec, i):
    idx = jnp.full((L, 1), i, jnp.int32)
    return lax.gather(vec, idx, _GDN, slice_sizes=(1,),
                      mode=lax.GatherScatterMode.PROMISE_IN_BOUNDS)


def _body(node_ref, ei0_ref, ei1_ref, rid_ref, rel_ref, proj_ref, out_ref,
          head0, tail0, proj0, sem0,
          head1, tail1, proj1, sem1,
          hidx_a, tidx_a, ridx_a, rel_tab, score_m,
          out_v0, osem0, out_v1, osem1):
    wid = lax.axis_index("s") * NC + lax.axis_index("c")
    ngroups = jnp.where(wid < L, TMAX, TMAX - 1)
    g0 = wid * (TMAX - 1) + jnp.minimum(wid, L)
    e0 = g0 * G
    pe0 = jnp.minimum(e0, N_EDGES_TOTAL - TMAX * G)
    off0 = e0 - pe0
    pltpu.sync_copy(ei0_ref.at[pl.ds(pe0, TMAX * G)], hidx_a)
    pltpu.sync_copy(ei1_ref.at[pl.ds(pe0, TMAX * G)], tidx_a)
    pltpu.sync_copy(rid_ref.at[pl.ds(pe0, TMAX * G)], ridx_a)
    pltpu.sync_copy(rel_ref, rel_tab)
    lanes = lax.iota(jnp.int32, L)

    bufs = ((head0, tail0, proj0, sem0), (head1, tail1, proj1, sem1))
    obufs = ((out_v0, osem0), (out_v1, osem1))

    def issue(t, buf):
        head_v, tail_v, proj_v, sem = buf

        @pl.when(t < ngroups)
        def _():
            hv = hidx_a[pl.ds(off0 + t * G, G)]
            tv = tidx_a[pl.ds(off0 + t * G, G)]
            rv = ridx_a[pl.ds(off0 + t * G, G)]
            pltpu.async_copy(node_ref.at[hv], head_v, sem)
            pltpu.async_copy(node_ref.at[tv], tail_v, sem)
            pltpu.async_copy(proj_ref.at[rv], proj_v, sem)

    def compute(t, buf, obuf):
        head_v, tail_v, proj_v, sem = buf
        out_v, osem = obuf

        @pl.when(t < ngroups)
        def _():
            @pl.when(t >= 2)
            def _():
                pltpu.make_async_copy(out_v, out_ref.at[pl.ds(e0, G)],
                                      osem).wait()
            pltpu.make_async_copy(node_ref.at[hidx_a.at[pl.ds(0, G)]],
                                  head_v, sem).wait()
            pltpu.make_async_copy(node_ref.at[hidx_a.at[pl.ds(0, G)]],
                                  tail_v, sem).wait()
            pltpu.make_async_copy(proj_ref.at[ridx_a.at[pl.ds(0, G)]],
                                  proj_v, sem).wait()
            rvec = ridx_a[pl.ds(off0 + t * G, G)]

            @plsc.parallel_loop(0, G, 1, unroll=4)
            def edge(e):
                rb = _bcast_lane(rvec, e)
                accs = [plsc.load_gather(rel_tab, [rb * RD + lanes])]
                accs += [jnp.zeros((L,), jnp.float32) for _ in range(7)]
                for c in range(ED // L):
                    dv = (head_v[e, pl.ds(c * L, L)]
                          - tail_v[e, pl.ds(c * L, L)])
                    for m in range(L // 2):
                        pw = proj_v[e, pl.ds((c * (L // 2) + m) * RD, RD)]
                        pa, pb = plsc.unpack(
                            plsc.bitcast(pw, jnp.bfloat16),
                            format=plsc.PackFormat.INTERLEAVED)
                        k = (2 * m) % 8
                        accs[k] = accs[k] + _bcast_lane(dv, 2 * m) * pa
                        accs[k + 1] = (accs[k + 1]
                                       + _bcast_lane(dv, 2 * m + 1) * pb)
                a0 = (accs[0] + accs[2]) + (accs[4] + accs[6])
                a1 = (accs[1] + accs[3]) + (accs[5] + accs[7])
                plsc.store_scatter(
                    score_m, [jnp.full((L,), e * SM, jnp.int32) + lanes],
                    jnp.full((L,), GAMMA / L, jnp.float32)
                    - jnp.abs(a0 + a1))

            rows = lanes * SM
            sv = plsc.load_gather(score_m, [rows])
            for j in range(1, L):
                sv = sv + plsc.load_gather(
                    score_m, [rows + jnp.full((L,), j, jnp.int32)])
            out_v[...] = sv
            pltpu.async_copy(out_v, out_ref.at[pl.ds(e0 + t * G, G)], osem)

    issue(0, bufs[0])

    def pair(p, carry):
        t = p * 2
        issue(t + 1, bufs[1])
        compute(t, bufs[0], obufs[0])
        issue(t + 2, bufs[0])
        compute(t + 1, bufs[1], obufs[1])
        return carry

    lax.fori_loop(0, (TMAX + 1) // 2, pair, 0, unroll=False)
    pltpu.make_async_copy(out_v0, out_ref.at[pl.ds(e0, G)], osem0).wait()
    pltpu.make_async_copy(out_v1, out_ref.at[pl.ds(e0, G)], osem1).wait()


@jax.jit
def _sc_call(node_emb, edge_index, rel_id, rel_emb_table, projection_table):
    mesh = plsc.VectorSubcoreMesh(core_axis_name="c", subcore_axis_name="s")
    staging = [
        pltpu.VMEM((G, ED), jnp.float32),
        pltpu.VMEM((G, ED), jnp.float32),
        pltpu.VMEM((G, ED * RD // 2), jnp.int32),
        pltpu.SemaphoreType.DMA,
    ]
    f = pl.kernel(
        _body,
        out_type=jax.ShapeDtypeStruct((N_EDGES_TOTAL,), jnp.float32),
        mesh=mesh,
        scratch_types=staging + staging + [
            pltpu.VMEM((TMAX * G,), jnp.int32),
            pltpu.VMEM((TMAX * G,), jnp.int32),
            pltpu.VMEM((TMAX * G,), jnp.int32),
            pltpu.VMEM((1000 * RD,), jnp.float32),
            pltpu.VMEM((G * SM,), jnp.float32),
            pltpu.VMEM((G,), jnp.float32),
            pltpu.SemaphoreType.DMA,
            pltpu.VMEM((G,), jnp.float32),
            pltpu.SemaphoreType.DMA,
        ],
        compiler_params=pltpu.CompilerParams(needs_layout_passes=False),
    )
    proj_bf = (projection_table.reshape(-1, ED // 2, 2, RD)
               .swapaxes(2, 3).astype(jnp.bfloat16)
               .reshape(-1, ED * RD // 2, 2))
    proj_i32 = lax.bitcast_convert_type(proj_bf, jnp.int32)
    return f(node_emb, edge_index[0], edge_index[1], rel_id,
             rel_emb_table.reshape(-1), proj_i32)


def kernel(node_emb, edge_index, rel_id, rel_emb_table, projection_table):
    return _sc_call(node_emb, edge_index, rel_id, rel_emb_table,
                    projection_table)


# speedup vs baseline: 2.4353x; 1.0058x over previous
import jax
import jax.numpy as jnp
from jax import lax
from jax.experimental import pallas as pl
from jax.experimental.pallas import tpu as pltpu
from jax.experimental.pallas import tpu_sc as plsc

GAMMA = 12.0
N_EDGES_TOTAL = 160000
ED = 128
RD = 16
L = 16
NC = 2
NS = 16
NW = NC * NS
G = 16
NGROUPS = N_EDGES_TOTAL // G
TMAX = (NGROUPS + NW - 1) // NW
SM = 17

_GDN = lax.GatherDimensionNumbers(
    offset_dims=(), collapsed_slice_dims=(0,), start_index_map=(0,))


def _bcast_lane(vec, i):
    idx = jnp.full((L, 1), i, jnp.int32)
    return lax.gather(vec, idx, _GDN, slice_sizes=(1,),
                      mode=lax.GatherScatterMode.PROMISE_IN_BOUNDS)


def _body(node_ref, ei0_ref, ei1_ref, rid_ref, rel_ref, proj_ref, out_ref,
          head0, tail0, proj0, sem0,
          head1, tail1, proj1, sem1,
          hidx_a, tidx_a, ridx_a, rel_tab, score_m,
          out_v0, osem0, out_v1, osem1):
    wid = lax.axis_index("s") * NC + lax.axis_index("c")
    ngroups = jnp.where(wid < L, TMAX, TMAX - 1)
    g0 = wid * (TMAX - 1) + jnp.minimum(wid, L)
    e0 = g0 * G
    pe0 = jnp.minimum(e0, N_EDGES_TOTAL - TMAX * G)
    off0 = e0 - pe0
    pltpu.sync_copy(ei0_ref.at[pl.ds(pe0, TMAX * G)], hidx_a)
    pltpu.sync_copy(ei1_ref.at[pl.ds(pe0, TMAX * G)], tidx_a)
    pltpu.sync_copy(rid_ref.at[pl.ds(pe0, TMAX * G)], ridx_a)
    pltpu.sync_copy(rel_ref, rel_tab)
    lanes = lax.iota(jnp.int32, L)

    bufs = ((head0, tail0, proj0, sem0), (head1, tail1, proj1, sem1))
    obufs = ((out_v0, osem0), (out_v1, osem1))

    def issue(t, buf):
        head_v, tail_v, proj_v, sem = buf

        @pl.when(t < ngroups)
        def _():
            hv = hidx_a[pl.ds(off0 + t * G, G)]
            tv = tidx_a[pl.ds(off0 + t * G, G)]
            rv = ridx_a[pl.ds(off0 + t * G, G)]
            pltpu.async_copy(node_ref.at[hv], head_v, sem)
            pltpu.async_copy(node_ref.at[tv], tail_v, sem)
            pltpu.async_copy(proj_ref.at[rv], proj_v, sem)

    def compute(t, buf, obuf):
        head_v, tail_v, proj_v, sem = buf
        out_v, osem = obuf

        @pl.when(t < ngroups)
        def _():
            @pl.when(t >= 2)
            def _():
                pltpu.make_async_copy(out_v, out_ref.at[pl.ds(e0, G)],
                                      osem).wait()
            pltpu.make_async_copy(node_ref.at[hidx_a.at[pl.ds(0, G)]],
                                  head_v, sem).wait()
            pltpu.make_async_copy(node_ref.at[hidx_a.at[pl.ds(0, G)]],
                                  tail_v, sem).wait()
            pltpu.make_async_copy(proj_ref.at[ridx_a.at[pl.ds(0, G)]],
                                  proj_v, sem).wait()
            rvec = ridx_a[pl.ds(off0 + t * G, G)]

            @plsc.parallel_loop(0, G, 1, unroll=4)
            def edge(e):
                rb = _bcast_lane(rvec, e)
                accs = [plsc.load_gather(rel_tab, [rb * RD + lanes])]
                accs += [jnp.zeros((L,), jnp.float32) for _ in range(3)]
                for c in range(ED // L):
                    dv = (head_v[e, pl.ds(c * L, L)]
                          - tail_v[e, pl.ds(c * L, L)])
                    for m in range(L // 2):
                        pw = proj_v[e, pl.ds((c * (L // 2) + m) * RD, RD)]
                        pa, pb = plsc.unpack(
                            plsc.bitcast(pw, jnp.bfloat16),
                            format=plsc.PackFormat.INTERLEAVED)
                        k = (2 * m) % 4
                        accs[k] = accs[k] + _bcast_lane(dv, 2 * m) * pa
                        accs[k + 1] = (accs[k + 1]
                                       + _bcast_lane(dv, 2 * m + 1) * pb)
                a0 = accs[0] + accs[2]
                a1 = accs[1] + accs[3]
                plsc.store_scatter(
                    score_m, [jnp.full((L,), e * SM, jnp.int32) + lanes],
                    jnp.full((L,), GAMMA / L, jnp.float32)
                    - jnp.abs(a0 + a1))

            rows = lanes * SM
            sv = plsc.load_gather(score_m, [rows])
            for j in range(1, L):
                sv = sv + plsc.load_gather(
                    score_m, [rows + jnp.full((L,), j, jnp.int32)])
            out_v[...] = sv
            pltpu.async_copy(out_v, out_ref.at[pl.ds(e0 + t * G, G)], osem)

    issue(0, bufs[0])

    def pair(p, carry):
        t = p * 2
        issue(t + 1, bufs[1])
        compute(t, bufs[0], obufs[0])
        issue(t + 2, bufs[0])
        compute(t + 1, bufs[1], obufs[1])
        return carry

    lax.fori_loop(0, (TMAX + 1) // 2, pair, 0, unroll=False)
    pltpu.make_async_copy(out_v0, out_ref.at[pl.ds(e0, G)], osem0).wait()
    pltpu.make_async_copy(out_v1, out_ref.at[pl.ds(e0, G)], osem1).wait()


@jax.jit
def _sc_call(node_emb, edge_index, rel_id, rel_emb_table, projection_table):
    mesh = plsc.VectorSubcoreMesh(core_axis_name="c", subcore_axis_name="s")
    staging = [
        pltpu.VMEM((G, ED), jnp.float32),
        pltpu.VMEM((G, ED), jnp.float32),
        pltpu.VMEM((G, ED * RD // 2), jnp.int32),
        pltpu.SemaphoreType.DMA,
    ]
    f = pl.kernel(
        _body,
        out_type=jax.ShapeDtypeStruct((N_EDGES_TOTAL,), jnp.float32),
        mesh=mesh,
        scratch_types=staging + staging + [
            pltpu.VMEM((TMAX * G,), jnp.int32),
            pltpu.VMEM((TMAX * G,), jnp.int32),
            pltpu.VMEM((TMAX * G,), jnp.int32),
            pltpu.VMEM((1000 * RD,), jnp.float32),
            pltpu.VMEM((G * SM,), jnp.float32),
            pltpu.VMEM((G,), jnp.float32),
            pltpu.SemaphoreType.DMA,
            pltpu.VMEM((G,), jnp.float32),
            pltpu.SemaphoreType.DMA,
        ],
        compiler_params=pltpu.CompilerParams(needs_layout_passes=False),
    )
    proj_bf = (projection_table.reshape(-1, ED // 2, 2, RD)
               .swapaxes(2, 3).astype(jnp.bfloat16)
               .reshape(-1, ED * RD // 2, 2))
    proj_i32 = lax.bitcast_convert_type(proj_bf, jnp.int32)
    return f(node_emb, edge_index[0], edge_index[1], rel_id,
             rel_emb_table.reshape(-1), proj_i32)


def kernel(node_emb, edge_index, rel_id, rel_emb_table, projection_table):
    return _sc_call(node_emb, edge_index, rel_id, rel_emb_table,
                    projection_table)
